# baseline (speedup 1.0000x reference)
import jax
import jax.numpy as jnp
from jax import lax
from jax.experimental import pallas as pl
from jax.experimental.pallas import tpu as pltpu
from jax.experimental.pallas import tpu_sc as plsc

NC, NS, LANES = 2, 16, 16
NW = NC * NS
EMBED = 32
BBLK = 128


def _sc_body(ids_hbm, ctxi_hbm, base_hbm, sup_hbm, out_hbm,
             stage, idt, ctxt, brows, srows, obuf, sem_g, sem_o):
    nb, hist = ids_hbm.shape
    wid = lax.axis_index("s") * NC + lax.axis_index("c")
    b0 = wid * BBLK
    iota = lax.iota(jnp.int32, LANES)

    pltpu.sync_copy(ids_hbm.at[pl.ds(b0, BBLK)], stage)

    def t_ids(h, c):
        hv = jnp.full((LANES,), h, jnp.int32)
        for tb in range(BBLK // LANES):
            g = plsc.load_gather(stage, [iota + tb * LANES, hv])
            idt[h, pl.ds(tb * LANES, LANES)] = g
        return c

    lax.fori_loop(0, hist, t_ids, 0)

    pltpu.sync_copy(ctxi_hbm.at[pl.ds(b0, BBLK)], stage)

    def t_ctx(h, c):
        hv = jnp.full((LANES,), h, jnp.int32)
        for tb in range(BBLK // LANES):
            g = plsc.load_gather(stage, [iota + tb * LANES, hv])
            ctxt[h, pl.ds(tb * LANES, LANES)] = plsc.bitcast(g, jnp.float32)
        return c

    lax.fori_loop(0, hist, t_ctx, 0)

    pltpu.async_copy(base_hbm.at[idt.at[0]], brows.at[0], sem_g)
    pltpu.async_copy(sup_hbm.at[idt.at[0]], srows.at[0], sem_g)

    def main(h, c):
        p = lax.rem(h, 2)

        @pl.when(h < hist - 1)
        def _():
            pltpu.async_copy(base_hbm.at[idt.at[h + 1]], brows.at[1 - p], sem_g)
            pltpu.async_copy(sup_hbm.at[idt.at[h + 1]], srows.at[1 - p], sem_g)

        pltpu.make_async_copy(base_hbm.at[idt.at[h]], brows.at[p], sem_g).wait()
        pltpu.make_async_copy(sup_hbm.at[idt.at[h]], srows.at[p], sem_g).wait()

        @pl.when(h >= 2)
        def _():
            pltpu.make_async_copy(
                obuf.at[p], out_hbm.at[0, :, pl.ds(b0, BBLK)], sem_o).wait()

        for tb in range(BBLK // LANES):
            cv = ctxt[h, pl.ds(tb * LANES, LANES)]
            rowv = iota + tb * LANES
            for d in range(EMBED):
                dv = jnp.full((LANES,), d, jnp.int32)
                bg = plsc.load_gather(brows.at[p], [rowv, dv])
                sg = plsc.load_gather(srows.at[p], [rowv, dv])
                obuf[p, d, pl.ds(tb * LANES, LANES)] = bg + cv * sg

        pltpu.async_copy(obuf.at[p], out_hbm.at[h, :, pl.ds(b0, BBLK)], sem_o)
        return c

    lax.fori_loop(0, hist, main, 0)

    pltpu.make_async_copy(obuf.at[0], out_hbm.at[0, :, pl.ds(b0, BBLK)], sem_o).wait()
    pltpu.make_async_copy(obuf.at[1], out_hbm.at[1, :, pl.ds(b0, BBLK)], sem_o).wait()


def kernel(input_ids, context_vector, base_table, superposed_table):
    b, h = input_ids.shape
    ids = input_ids.astype(jnp.int32)
    ctxi = lax.bitcast_convert_type(context_vector, jnp.int32)
    mesh = plsc.VectorSubcoreMesh(core_axis_name="c", subcore_axis_name="s",
                                  num_cores=NC, num_subcores=NS)
    out_t = pl.kernel(
        _sc_body,
        out_type=jax.ShapeDtypeStruct((h, EMBED, b), jnp.float32),
        mesh=mesh,
        scratch_types=[
            pltpu.VMEM((BBLK, h), jnp.int32),
            pltpu.VMEM((h, BBLK), jnp.int32),
            pltpu.VMEM((h, BBLK), jnp.float32),
            pltpu.VMEM((2, BBLK, EMBED), jnp.float32),
            pltpu.VMEM((2, BBLK, EMBED), jnp.float32),
            pltpu.VMEM((2, EMBED, BBLK), jnp.float32),
            pltpu.SemaphoreType.DMA,
            pltpu.SemaphoreType.DMA,
        ],
        compiler_params=pltpu.CompilerParams(
            use_tc_tiling_on_sc=False, needs_layout_passes=False),
    )(ids, ctxi, base_table, superposed_table)
    return out_t.transpose(2, 0, 1)


# speedup vs baseline: 1.2474x; 1.2474x over previous
import jax
import jax.numpy as jnp
from jax import lax
from jax.experimental import pallas as pl
from jax.experimental.pallas import tpu as pltpu
from jax.experimental.pallas import tpu_sc as plsc

NC, NS, LANES = 2, 16, 16
NW = NC * NS
EMBED = 32
BBLK = 128


def _sc_body(ids_hbm, ctxi_hbm, base_hbm, sup_hbm, out_hbm,
             stage, idt, ctxt, brows, srows, obuf, sem_g, sem_o):
    nb, hist = ids_hbm.shape
    wid = lax.axis_index("s") * NC + lax.axis_index("c")
    b0 = wid * BBLK
    iota = lax.iota(jnp.int32, LANES)

    pltpu.sync_copy(ids_hbm.at[pl.ds(b0, BBLK)], stage)

    def t_ids(h, c):
        hv = jnp.full((LANES,), h, jnp.int32)
        for tb in range(BBLK // LANES):
            g = plsc.load_gather(stage, [iota + tb * LANES, hv])
            idt[h, pl.ds(tb * LANES, LANES)] = g
        return c

    lax.fori_loop(0, hist, t_ids, 0)

    pltpu.sync_copy(ctxi_hbm.at[pl.ds(b0, BBLK)], stage)

    def t_ctx(h, c):
        hv = jnp.full((LANES,), h, jnp.int32)
        for tb in range(BBLK // LANES):
            g = plsc.load_gather(stage, [iota + tb * LANES, hv])
            ctxt[h, pl.ds(tb * LANES, LANES)] = plsc.bitcast(g, jnp.float32)
        return c

    lax.fori_loop(0, hist, t_ctx, 0)

    pltpu.async_copy(base_hbm.at[idt.at[0]], brows.at[0], sem_g)
    pltpu.async_copy(sup_hbm.at[idt.at[0]], srows.at[0], sem_g)

    def main(h, c):
        p = lax.rem(h, 2)

        @pl.when(h < hist - 1)
        def _():
            pltpu.async_copy(base_hbm.at[idt.at[h + 1]], brows.at[1 - p], sem_g)
            pltpu.async_copy(sup_hbm.at[idt.at[h + 1]], srows.at[1 - p], sem_g)

        pltpu.make_async_copy(base_hbm.at[idt.at[h]], brows.at[p], sem_g).wait()
        pltpu.make_async_copy(sup_hbm.at[idt.at[h]], srows.at[p], sem_g).wait()

        @pl.when(h >= 2)
        def _():
            pltpu.make_async_copy(
                obuf.at[p], out_hbm.at[0, :, pl.ds(b0, BBLK)], sem_o).wait()

        for tb in range(BBLK // LANES):
            cv = ctxt[h, pl.ds(tb * LANES, LANES)]
            for j in range(LANES):
                t = tb * LANES + j
                cb = jnp.full((LANES,), cv[j])
                tv = jnp.full((LANES,), t, jnp.int32)
                for hh in range(EMBED // LANES):
                    dsl = pl.ds(hh * LANES, LANES)
                    v = brows[p, t, dsl] + cb * srows[p, t, dsl]
                    plsc.store_scatter(obuf.at[p], [iota + hh * LANES, tv], v)

        pltpu.async_copy(obuf.at[p], out_hbm.at[h, :, pl.ds(b0, BBLK)], sem_o)
        return c

    lax.fori_loop(0, hist, main, 0)

    pltpu.make_async_copy(obuf.at[0], out_hbm.at[0, :, pl.ds(b0, BBLK)], sem_o).wait()
    pltpu.make_async_copy(obuf.at[1], out_hbm.at[1, :, pl.ds(b0, BBLK)], sem_o).wait()


def kernel(input_ids, context_vector, base_table, superposed_table):
    b, h = input_ids.shape
    ids = input_ids.astype(jnp.int32)
    ctxi = lax.bitcast_convert_type(context_vector, jnp.int32)
    mesh = plsc.VectorSubcoreMesh(core_axis_name="c", subcore_axis_name="s",
                                  num_cores=NC, num_subcores=NS)
    out_t = pl.kernel(
        _sc_body,
        out_type=jax.ShapeDtypeStruct((h, EMBED, b), jnp.float32),
        mesh=mesh,
        scratch_types=[
            pltpu.VMEM((BBLK, h), jnp.int32),
            pltpu.VMEM((h, BBLK), jnp.int32),
            pltpu.VMEM((h, BBLK), jnp.float32),
            pltpu.VMEM((2, BBLK, EMBED), jnp.float32),
            pltpu.VMEM((2, BBLK, EMBED), jnp.float32),
            pltpu.VMEM((2, EMBED, BBLK), jnp.float32),
            pltpu.SemaphoreType.DMA,
            pltpu.SemaphoreType.DMA,
        ],
        compiler_params=pltpu.CompilerParams(
            use_tc_tiling_on_sc=False, needs_layout_passes=False),
    )(ids, ctxi, base_table, superposed_table)
    return out_t.transpose(2, 0, 1)


# speedup vs baseline: 1.4323x; 1.1482x over previous
import jax
import jax.numpy as jnp
from jax import lax
from jax.experimental import pallas as pl
from jax.experimental.pallas import tpu as pltpu
from jax.experimental.pallas import tpu_sc as plsc

NC, NS, LANES = 2, 16, 16
NW = NC * NS
EMBED = 32
BBLK = 128
NBUF = 4


def _sc_body(ids_hbm, ctxi_hbm, base_hbm, sup_hbm, out_hbm,
             stage, idt, ctxt, brows, srows, obuf, sem_g, sem_o):
    nb, hist = ids_hbm.shape
    wid = lax.axis_index("s") * NC + lax.axis_index("c")
    b0 = wid * BBLK
    iota = lax.iota(jnp.int32, LANES)

    pltpu.sync_copy(ids_hbm.at[pl.ds(b0, BBLK)], stage)

    def t_ids(h, c):
        hv = jnp.full((LANES,), h, jnp.int32)
        for tb in range(BBLK // LANES):
            g = plsc.load_gather(stage, [iota + tb * LANES, hv])
            idt[h, pl.ds(tb * LANES, LANES)] = g
        return c

    lax.fori_loop(0, hist, t_ids, 0)

    pltpu.sync_copy(ctxi_hbm.at[pl.ds(b0, BBLK)], stage)

    def t_ctx(h, c):
        hv = jnp.full((LANES,), h, jnp.int32)
        for tb in range(BBLK // LANES):
            g = plsc.load_gather(stage, [iota + tb * LANES, hv])
            ctxt[h, pl.ds(tb * LANES, LANES)] = plsc.bitcast(g, jnp.float32)
        return c

    lax.fori_loop(0, hist, t_ctx, 0)

    for hp in range(NBUF - 1):
        pltpu.async_copy(base_hbm.at[idt.at[hp]], brows.at[hp], sem_g)
        pltpu.async_copy(sup_hbm.at[idt.at[hp]], srows.at[hp], sem_g)

    def main(h, c):
        p = lax.rem(h, NBUF)

        @pl.when(h < hist - (NBUF - 1))
        def _():
            q = lax.rem(h + NBUF - 1, NBUF)
            pltpu.async_copy(base_hbm.at[idt.at[h + NBUF - 1]], brows.at[q], sem_g)
            pltpu.async_copy(sup_hbm.at[idt.at[h + NBUF - 1]], srows.at[q], sem_g)

        pltpu.make_async_copy(base_hbm.at[idt.at[h]], brows.at[p], sem_g).wait()
        pltpu.make_async_copy(sup_hbm.at[idt.at[h]], srows.at[p], sem_g).wait()

        po = lax.rem(h, 2)

        @pl.when(h >= 2)
        def _():
            pltpu.make_async_copy(
                obuf.at[po], out_hbm.at[0, :, pl.ds(b0, BBLK)], sem_o).wait()

        ds0 = pl.ds(0, LANES)
        ds1 = pl.ds(LANES, LANES)
        row0 = iota
        row1 = iota + LANES
        for tb in range(BBLK // LANES):
            cv = ctxt[h, pl.ds(tb * LANES, LANES)]
            for j in range(0, LANES, 2):
                t0 = tb * LANES + j
                t1 = t0 + 1
                cb0 = jnp.full((LANES,), cv[j])
                cb1 = jnp.full((LANES,), cv[j + 1])
                b00 = brows[p, t0, ds0]
                s00 = srows[p, t0, ds0]
                b01 = brows[p, t0, ds1]
                s01 = srows[p, t0, ds1]
                b10 = brows[p, t1, ds0]
                s10 = srows[p, t1, ds0]
                b11 = brows[p, t1, ds1]
                s11 = srows[p, t1, ds1]
                v00 = b00 + cb0 * s00
                v01 = b01 + cb0 * s01
                v10 = b10 + cb1 * s10
                v11 = b11 + cb1 * s11
                tv0 = jnp.full((LANES,), t0, jnp.int32)
                tv1 = jnp.full((LANES,), t1, jnp.int32)
                plsc.store_scatter(obuf.at[po], [row0, tv0], v00)
                plsc.store_scatter(obuf.at[po], [row1, tv0], v01)
                plsc.store_scatter(obuf.at[po], [row0, tv1], v10)
                plsc.store_scatter(obuf.at[po], [row1, tv1], v11)

        pltpu.async_copy(obuf.at[po], out_hbm.at[h, :, pl.ds(b0, BBLK)], sem_o)
        return c

    lax.fori_loop(0, hist, main, 0)

    pltpu.make_async_copy(obuf.at[0], out_hbm.at[0, :, pl.ds(b0, BBLK)], sem_o).wait()
    pltpu.make_async_copy(obuf.at[1], out_hbm.at[1, :, pl.ds(b0, BBLK)], sem_o).wait()


def kernel(input_ids, context_vector, base_table, superposed_table):
    b, h = input_ids.shape
    ids = input_ids.astype(jnp.int32)
    ctxi = lax.bitcast_convert_type(context_vector, jnp.int32)
    mesh = plsc.VectorSubcoreMesh(core_axis_name="c", subcore_axis_name="s",
                                  num_cores=NC, num_subcores=NS)
    out_t = pl.kernel(
        _sc_body,
        out_type=jax.ShapeDtypeStruct((h, EMBED, b), jnp.float32),
        mesh=mesh,
        scratch_types=[
            pltpu.VMEM((BBLK, h), jnp.int32),
            pltpu.VMEM((h, BBLK), jnp.int32),
            pltpu.VMEM((h, BBLK), jnp.float32),
            pltpu.VMEM((NBUF, BBLK, EMBED), jnp.float32),
            pltpu.VMEM((NBUF, BBLK, EMBED), jnp.float32),
            pltpu.VMEM((2, EMBED, BBLK), jnp.float32),
            pltpu.SemaphoreType.DMA,
            pltpu.SemaphoreType.DMA,
        ],
        compiler_params=pltpu.CompilerParams(
            use_tc_tiling_on_sc=False, needs_layout_passes=False),
    )(ids, ctxi, base_table, superposed_table)
    return out_t.transpose(2, 0, 1)
